# baseline (speedup 1.0000x reference)
import jax
import jax.numpy as jnp
from jax.experimental import pallas as pl
import jax.experimental.pallas.tpu as pltpu

_TOKENS = 2048
_D_IN = 768
_BASE = 128
_BT = 256


def _xs_kernel(x_ref, w_ref, b_ref, o_ref):
    o_ref[:] = jax.lax.dot_general(
        x_ref[:], w_ref[:],
        (((1,), (1,)), ((), ())),
        preferred_element_type=jnp.float32,
    ) + b_ref[:]


def _expand_kernel(xs_ref, o_ref):
    xs = xs_ref[:]
    x1 = xs[:, :_BASE]
    x2 = xs[:, _BASE:]
    for i in range(_BASE):
        o_ref[:, i * _BASE:(i + 1) * _BASE] = x1[:, i:i + 1] + x2


def kernel(x, W, b):
    b2 = b.reshape(1, 2 * _BASE)
    xs = pl.pallas_call(
        _xs_kernel,
        grid=(1,),
        in_specs=[
            pl.BlockSpec((_TOKENS, _D_IN), lambda i: (0, 0)),
            pl.BlockSpec((2 * _BASE, _D_IN), lambda i: (0, 0)),
            pl.BlockSpec((1, 2 * _BASE), lambda i: (0, 0)),
        ],
        out_specs=pl.BlockSpec((_TOKENS, 2 * _BASE), lambda i: (0, 0)),
        out_shape=jax.ShapeDtypeStruct((_TOKENS, 2 * _BASE), jnp.float32),
    )(x, W, b2)
    return pl.pallas_call(
        _expand_kernel,
        grid=(_TOKENS // _BT,),
        in_specs=[
            pl.BlockSpec((_BT, 2 * _BASE), lambda t: (t, 0)),
        ],
        out_specs=pl.BlockSpec((_BT, _BASE * _BASE), lambda t: (t, 0)),
        out_shape=jax.ShapeDtypeStruct((_TOKENS, _BASE * _BASE), jnp.float32),
        compiler_params=pltpu.CompilerParams(
            dimension_semantics=("parallel",),
        ),
    )(xs)
